# baseline (speedup 1.0000x reference)
import jax
import jax.numpy as jnp
from jax.experimental import pallas as pl
from jax.experimental.pallas import tpu as pltpu


def _conv_body(x_ref, w1_ref, b1_ref, w2_ref, b2_ref, w3_ref, b3_ref, o_ref):
    m = x_ref.shape[0]
    cdt = x_ref.dtype
    x0 = x_ref[...]

    p1 = jnp.concatenate(
        [x0, jnp.roll(x0, -1, 0), jnp.roll(x0, -11, 0), jnp.roll(x0, -12, 0)],
        axis=1)
    h = jnp.dot(p1, w1_ref[...], preferred_element_type=jnp.float32)
    h = jnp.maximum(h + b1_ref[...], 0.0).astype(cdt)

    p2 = jnp.concatenate(
        [h, jnp.roll(h, -1, 0), jnp.roll(h, -11, 0), jnp.roll(h, -12, 0)],
        axis=1)
    h = jnp.dot(p2, w2_ref[...], preferred_element_type=jnp.float32)
    h = jnp.maximum(h + b2_ref[...], 0.0).astype(cdt)

    zpad = jnp.zeros((m, 64), cdt)
    h2x = jnp.concatenate(
        [h, jnp.roll(h, -1, 0), jnp.roll(h, -2, 0), zpad], axis=1)
    p3 = jnp.concatenate(
        [h2x, jnp.roll(h2x, -11, 0), jnp.roll(h2x, -22, 0)], axis=1)
    h = jnp.dot(p3, w3_ref[...], preferred_element_type=jnp.float32)
    h = jnp.maximum(h + b3_ref[...], 0.0).astype(cdt)
    o_ref[...] = h


def _fc_body(x_ref, fw1_ref, fb1_ref, fw2_ref, fb2_ref, o_ref):
    cdt = x_ref.dtype
    h = jnp.dot(x_ref[...], fw1_ref[...], preferred_element_type=jnp.float32)
    h = jnp.maximum(h + fb1_ref[...], 0.0).astype(cdt)
    o = jnp.dot(h, fw2_ref[...], preferred_element_type=jnp.float32)
    o_ref[...] = (o + fb2_ref[...]).astype(o_ref.dtype)


def _repack_conv1(c1_w):
    f = jnp.arange(1024)[:, None]
    n = jnp.arange(128)[None, :]
    tr, tc = (f // 256) // 2, (f // 256) % 2
    c, u, v = (f // 64) % 4, (f // 8) % 8, f % 8
    py, px, co = n // 64, (n // 32) % 2, n % 32
    ki = 8 * tr + u - 4 * py
    kj = 8 * tc + v - 4 * px
    valid = (ki >= 0) & (ki < 8) & (kj >= 0) & (kj < 8)
    src = c * 64 + jnp.clip(ki, 0, 7) * 8 + jnp.clip(kj, 0, 7)
    return jnp.where(valid, c1_w[src, jnp.broadcast_to(co, src.shape)], 0.0)


def _repack_conv2(c2_w):
    n = jnp.arange(512)
    i_w, j_w = n // 256, (n // 128) % 2
    py, px, c = (n // 64) % 2, (n // 32) % 2, n % 32
    src = c * 16 + (2 * i_w + py) * 4 + (2 * j_w + px)
    return c2_w[src]


def _repack_conv3(c3_w):
    f = jnp.arange(768)[:, None]
    co = jnp.arange(64)[None, :]
    i_w, j_w, c = f // 256, (f // 64) % 4, f % 64
    valid = j_w < 3
    src = c * 9 + i_w * 3 + jnp.clip(j_w, 0, 2)
    return jnp.where(valid, c3_w[src, co], 0.0)


@jax.jit
def kernel(c1_w, c1_b, c2_w, c2_b, c3_w, c3_b, fc1_w, fc1_b, fc2_w, fc2_b, x):
    B = x.shape[0]
    bf = jnp.bfloat16

    w1p = _repack_conv1(c1_w).astype(bf)
    b1p = jnp.tile(c1_b, (1, 4))
    w2p = _repack_conv2(c2_w).astype(bf)
    w3p = _repack_conv3(c3_w).astype(bf)
    fw1 = fc1_w.astype(bf)
    fw2 = fc2_w.astype(bf)

    xoct = jax.lax.conv_general_dilated_patches(
        x.astype(bf), (8, 8), (8, 8), [(0, 4), (0, 4)],
        dimension_numbers=("NCHW", "HWIO", "NHWC"),
    ).reshape(B * 121, 256)

    bt = 32 if B % 32 == 0 else B
    h3 = pl.pallas_call(
        _conv_body,
        out_shape=jax.ShapeDtypeStruct((B * 121, 64), bf),
        grid=(B // bt,),
        in_specs=[
            pl.BlockSpec((bt * 121, 256), lambda i: (i, 0)),
            pl.BlockSpec((1024, 128), lambda i: (0, 0)),
            pl.BlockSpec((1, 128), lambda i: (0, 0)),
            pl.BlockSpec((512, 64), lambda i: (0, 0)),
            pl.BlockSpec((1, 64), lambda i: (0, 0)),
            pl.BlockSpec((768, 64), lambda i: (0, 0)),
            pl.BlockSpec((1, 64), lambda i: (0, 0)),
        ],
        out_specs=pl.BlockSpec((bt * 121, 64), lambda i: (i, 0)),
        compiler_params=pltpu.CompilerParams(
            dimension_semantics=("parallel",),
            vmem_limit_bytes=100 * 1024 * 1024,
        ),
    )(xoct, w1p, b1p, w2p, c2_b, w3p, c3_b)

    hf = h3.reshape(B, 11, 11, 64)[:, :7, :7, :].reshape(B, 49 * 64)

    btf = 256 if B % 256 == 0 else B
    out = pl.pallas_call(
        _fc_body,
        out_shape=jax.ShapeDtypeStruct((B, 128), jnp.float32),
        grid=(B // btf,),
        in_specs=[
            pl.BlockSpec((btf, 3136), lambda i: (i, 0)),
            pl.BlockSpec((3136, 512), lambda i: (0, 0)),
            pl.BlockSpec((1, 512), lambda i: (0, 0)),
            pl.BlockSpec((512, 128), lambda i: (0, 0)),
            pl.BlockSpec((1, 128), lambda i: (0, 0)),
        ],
        out_specs=pl.BlockSpec((btf, 128), lambda i: (i, 0)),
        compiler_params=pltpu.CompilerParams(
            dimension_semantics=("parallel",),
            vmem_limit_bytes=100 * 1024 * 1024,
        ),
    )(hf, fw1, fc1_b, fw2, fc2_b)
    return out[:, :18]


# speedup vs baseline: 1.9838x; 1.9838x over previous
import jax
import jax.numpy as jnp
from jax.experimental import pallas as pl
from jax.experimental.pallas import tpu as pltpu


def _conv_body(xee_ref, xeo_ref, xoe_ref, xoo_ref,
               w1_ref, b1_ref, w2_ref, b2_ref, w3_ref, b3_ref, o_ref):
    bt = xee_ref.shape[0]
    cdt = xee_ref.dtype
    planes = {(0, 0): xee_ref[...], (0, 1): xeo_ref[...],
              (1, 0): xoe_ref[...], (1, 1): xoo_ref[...]}

    parts = [
        planes[(i % 2, j % 2)][:, i // 2:i // 2 + 10, j // 2:j // 2 + 10, :]
        for i in range(3) for j in range(3)
    ]
    p1 = jnp.concatenate(parts, axis=3).reshape(bt * 100, 576)
    h = jnp.dot(p1, w1_ref[...], preferred_element_type=jnp.float32)
    h = jnp.maximum(h + b1_ref[...], 0.0).astype(cdt)
    h = h.reshape(bt, 10, 10, 128)

    parts = [h[:, i:i + 9, j:j + 9, :] for i in range(2) for j in range(2)]
    p2 = jnp.concatenate(parts, axis=3).reshape(bt * 81, 512)
    h = jnp.dot(p2, w2_ref[...], preferred_element_type=jnp.float32)
    h = jnp.maximum(h + b2_ref[...], 0.0).astype(cdt)
    h = h.reshape(bt, 9, 9, 64)

    parts = [h[:, i:i + 7, j:j + 7, :] for i in range(3) for j in range(3)]
    p3 = jnp.concatenate(parts, axis=3).reshape(bt * 49, 576)
    h = jnp.dot(p3, w3_ref[...], preferred_element_type=jnp.float32)
    h = jnp.maximum(h + b3_ref[...], 0.0).astype(cdt)
    o_ref[...] = h


def _fc_body(x_ref, fw1_ref, fb1_ref, fw2_ref, fb2_ref, o_ref):
    cdt = x_ref.dtype
    h = jnp.dot(x_ref[...], fw1_ref[...], preferred_element_type=jnp.float32)
    h = jnp.maximum(h + fb1_ref[...], 0.0).astype(cdt)
    o = jnp.dot(h, fw2_ref[...], preferred_element_type=jnp.float32)
    o_ref[...] = (o + fb2_ref[...]).astype(o_ref.dtype)


def _repack_conv1(c1_w):
    f = jnp.arange(576)[:, None]
    n = jnp.arange(128)[None, :]
    i_w, j_w = f // 192, (f // 64) % 3
    c, dy, dx = (f // 16) % 4, (f // 4) % 4, f % 4
    py, px, co = n // 64, (n // 32) % 2, n % 32
    ki = (i_w - py) * 4 + dy
    kj = (j_w - px) * 4 + dx
    valid = (ki >= 0) & (ki < 8) & (kj >= 0) & (kj < 8)
    src = c * 64 + jnp.clip(ki, 0, 7) * 8 + jnp.clip(kj, 0, 7)
    return jnp.where(valid, c1_w[src, jnp.broadcast_to(co, src.shape)], 0.0)


def _repack_conv2(c2_w):
    n = jnp.arange(512)
    i_w, j_w = n // 256, (n // 128) % 2
    py, px, c = (n // 64) % 2, (n // 32) % 2, n % 32
    src = c * 16 + (2 * i_w + py) * 4 + (2 * j_w + px)
    return c2_w[src]


def _repack_conv3(c3_w):
    m = jnp.arange(576)
    src = (m % 64) * 9 + (m // 192) * 3 + ((m // 64) % 3)
    return c3_w[src]


@jax.jit
def kernel(c1_w, c1_b, c2_w, c2_b, c3_w, c3_b, fc1_w, fc1_b, fc2_w, fc2_b, x):
    B = x.shape[0]
    bf = jnp.bfloat16

    w1p = _repack_conv1(c1_w).astype(bf)
    b1p = jnp.tile(c1_b, (1, 4))
    w2p = _repack_conv2(c2_w).astype(bf)
    w3p = _repack_conv3(c3_w).astype(bf)
    fw1 = fc1_w.astype(bf)
    fw2 = fc2_w.astype(bf)

    xp = jax.lax.conv_general_dilated_patches(
        x.astype(bf), (4, 4), (4, 4), "VALID",
        dimension_numbers=("NCHW", "HWIO", "NHWC"))
    xee = xp[:, 0::2, 0::2]
    xeo = xp[:, 0::2, 1::2]
    xoe = xp[:, 1::2, 0::2]
    xoo = xp[:, 1::2, 1::2]

    bt = 64 if B % 64 == 0 else B
    h3 = pl.pallas_call(
        _conv_body,
        out_shape=jax.ShapeDtypeStruct((B * 49, 64), bf),
        grid=(B // bt,),
        in_specs=[
            pl.BlockSpec((bt, 11, 11, 64), lambda i: (i, 0, 0, 0)),
            pl.BlockSpec((bt, 11, 10, 64), lambda i: (i, 0, 0, 0)),
            pl.BlockSpec((bt, 10, 11, 64), lambda i: (i, 0, 0, 0)),
            pl.BlockSpec((bt, 10, 10, 64), lambda i: (i, 0, 0, 0)),
            pl.BlockSpec((576, 128), lambda i: (0, 0)),
            pl.BlockSpec((1, 128), lambda i: (0, 0)),
            pl.BlockSpec((512, 64), lambda i: (0, 0)),
            pl.BlockSpec((1, 64), lambda i: (0, 0)),
            pl.BlockSpec((576, 64), lambda i: (0, 0)),
            pl.BlockSpec((1, 64), lambda i: (0, 0)),
        ],
        out_specs=pl.BlockSpec((bt * 49, 64), lambda i: (i, 0)),
        compiler_params=pltpu.CompilerParams(
            dimension_semantics=("parallel",),
            vmem_limit_bytes=100 * 1024 * 1024,
        ),
    )(xee, xeo, xoe, xoo, w1p, b1p, w2p, c2_b, w3p, c3_b)

    hf = h3.reshape(B, 49 * 64)

    btf = 256 if B % 256 == 0 else B
    out = pl.pallas_call(
        _fc_body,
        out_shape=jax.ShapeDtypeStruct((B, 128), jnp.float32),
        grid=(B // btf,),
        in_specs=[
            pl.BlockSpec((btf, 3136), lambda i: (i, 0)),
            pl.BlockSpec((3136, 512), lambda i: (0, 0)),
            pl.BlockSpec((1, 512), lambda i: (0, 0)),
            pl.BlockSpec((512, 128), lambda i: (0, 0)),
            pl.BlockSpec((1, 128), lambda i: (0, 0)),
        ],
        out_specs=pl.BlockSpec((btf, 128), lambda i: (i, 0)),
        compiler_params=pltpu.CompilerParams(
            dimension_semantics=("parallel",),
            vmem_limit_bytes=100 * 1024 * 1024,
        ),
    )(hf, fw1, fc1_b, fw2, fc2_b)
    return out[:, :18]
